# Initial kernel scaffold; baseline (speedup 1.0000x reference)
import jax
import jax.numpy as jnp
from jax.experimental import pallas as pl


def kernel(user_preference, item_preference, edge_values, edge_index):
    raise NotImplementedError("write your pallas kernel here")


# speedup vs baseline: 3.2237x; 3.2237x over previous
import functools

import jax
import jax.numpy as jnp
from jax import lax
from jax.experimental import pallas as pl
from jax.experimental.pallas import tpu as pltpu
from jax.experimental.pallas import tpu_sc as plsc

NU = 10000
NI = 10000
D = 128
L = 16
NSUB = 16
CHUNK = 128
ACC_ROWS = 10240


def _layer_body(item_hbm, user_hbm, rows_hbm, cols_hbm, vals_hbm,
                uout_hbm, iout_hbm,
                rows_v, sidx_v, didx_v, vals_v, acc_sh, sem):
    cid = lax.axis_index("c")
    sid = lax.axis_index("s")
    nchunks = rows_hbm.shape[0] // (NSUB * CHUNK)
    ept = nchunks * CHUNK

    def run_side(src_hbm, sidx_hbm, didx_hbm, out_hbm):
        def zrow(i, _):
            for j in range(D // L):
                rows_v[i, pl.ds(j * L, L)] = jnp.zeros((L,), jnp.float32)
            return 0
        lax.fori_loop(0, CHUNK, zrow, 0)
        rows_per_tile = ACC_ROWS // NSUB
        for k in range(rows_per_tile // CHUNK):
            pltpu.sync_copy(
                rows_v, acc_sh.at[pl.ds(sid * rows_per_tile + k * CHUNK, CHUNK)])
        plsc.subcore_barrier()

        base0 = sid * ept

        def chunk_body(g, _):
            b = base0 + g * CHUNK
            pltpu.sync_copy(sidx_hbm.at[pl.ds(b, CHUNK)], sidx_v)
            pltpu.sync_copy(didx_hbm.at[pl.ds(b, CHUNK)], didx_v)
            pltpu.sync_copy(vals_hbm.at[pl.ds(b, CHUNK)], vals_v.at[pl.ds(0, CHUNK)])
            pltpu.async_copy(src_hbm.at[sidx_v], rows_v, sem).wait()

            def ebody(e, _):
                s = vals_v[pl.ds(e, L)][0]
                for j in range(D // L):
                    sl = pl.ds(j * L, L)
                    rows_v[e, sl] = rows_v[e, sl] * s
                return 0
            lax.fori_loop(0, CHUNK, ebody, 0)

            pltpu.sync_copy(rows_v, acc_sh.at[didx_v], add=True)
            return 0
        lax.fori_loop(0, nchunks, chunk_body, 0)
        plsc.subcore_barrier()

        per = ACC_ROWS // NSUB
        row0 = sid * per
        for k in range(per // CHUNK):
            pltpu.sync_copy(acc_sh.at[pl.ds(row0 + k * CHUNK, CHUNK)],
                            out_hbm.at[pl.ds(row0 + k * CHUNK, CHUNK)])

    @pl.when(cid == 0)
    def _():
        run_side(item_hbm, cols_hbm, rows_hbm, uout_hbm)

    @pl.when(cid == 1)
    def _():
        run_side(user_hbm, rows_hbm, cols_hbm, iout_hbm)


@functools.cache
def _layer_fn():
    mesh = plsc.VectorSubcoreMesh(core_axis_name="c", subcore_axis_name="s")
    return pl.kernel(
        _layer_body,
        mesh=mesh,
        out_type=[
            jax.ShapeDtypeStruct((ACC_ROWS, D), jnp.float32),
            jax.ShapeDtypeStruct((ACC_ROWS, D), jnp.float32),
        ],
        scratch_types=[
            pltpu.VMEM((CHUNK, D), jnp.float32),
            pltpu.VMEM((CHUNK,), jnp.int32),
            pltpu.VMEM((CHUNK,), jnp.int32),
            pltpu.VMEM((CHUNK + L,), jnp.float32),
            pltpu.VMEM_SHARED((ACC_ROWS, D), jnp.float32),
            pltpu.SemaphoreType.DMA,
        ],
    )


def _avg_body(a, b, c, d, o):
    o[...] = (a[...] + b[...] + c[...] + d[...]) * 0.25


def _avg4(a, b, c, d, n):
    dd = a.shape[1]
    blk = 2000
    return pl.pallas_call(
        _avg_body,
        out_shape=jax.ShapeDtypeStruct((n, dd), jnp.float32),
        grid=(n // blk,),
        in_specs=[pl.BlockSpec((blk, dd), lambda i: (i, 0))] * 4,
        out_specs=pl.BlockSpec((blk, dd), lambda i: (i, 0)),
    )(a, b, c, d)


def kernel(user_preference, item_preference, edge_values, edge_index):
    rows = edge_index[0].astype(jnp.int32)
    cols = edge_index[1].astype(jnp.int32)
    vals = edge_values.astype(jnp.float32)
    n = rows.shape[0]
    ept = -(-n // (NSUB * CHUNK)) * CHUNK
    pad = NSUB * ept - n
    if pad:
        rows = jnp.concatenate([rows, jnp.zeros((pad,), jnp.int32)])
        cols = jnp.concatenate([cols, jnp.zeros((pad,), jnp.int32)])
        vals = jnp.concatenate([vals, jnp.zeros((pad,), jnp.float32)])

    layer = _layer_fn()
    u0, i0 = user_preference, item_preference
    u1, i1 = layer(i0, u0, rows, cols, vals)
    u2, i2 = layer(i1, u1, rows, cols, vals)
    u3, i3 = layer(i2, u2, rows, cols, vals)
    pu = _avg4(u0, u1, u2, u3, NU)
    pi = _avg4(i0, i1, i2, i3, NI)
    return pu, pi
